# baseline (speedup 1.0000x reference)
import functools

import jax
import jax.numpy as jnp
from jax import lax
from jax.experimental import pallas as pl
from jax.experimental.pallas import tpu as pltpu
from jax.experimental.pallas import tpu_sc as plsc

_T = 2048
_D = 1024
_E = 8
_H = 2752
_ALPHA = 0.05
_TILE = 128
_NPAD = _T + _E * _TILE
_NT = _NPAD // _TILE
_NW = 32


def _router_body(x_ref, wg_ref, bg_ref, dest_ref, inv_ref, ps_ref, eot_ref,
                 act_ref, aux_ref):
    x = x_ref[...]
    wg = wg_ref[...]
    logits = lax.dot_general(x, wg, (((1,), (1,)), ((), ())),
                             preferred_element_type=jnp.float32) + bg_ref[...]
    m = jnp.max(logits, axis=1, keepdims=True)
    ex = jnp.exp(logits - m)
    probs = ex / jnp.sum(ex, axis=1, keepdims=True)
    pmax = jnp.max(probs, axis=1, keepdims=True)
    lanes = lax.broadcasted_iota(jnp.int32, (_T, _E), 1)
    top1 = jnp.min(jnp.where(probs == pmax, lanes, _E), axis=1, keepdims=True)
    onehot = lanes == top1
    oh_f = onehot.astype(jnp.float32)
    counts = jnp.sum(oh_f, axis=0, keepdims=True)
    ce = jnp.sum(probs, axis=0, keepdims=True) * (1.0 / _T)
    me = counts * (1.0 / _T)
    aux_ref[...] = jnp.sum(me * ce, keepdims=True).reshape(1, 1) * (_ALPHA * _E)
    cs = oh_f
    k = 1
    while k < _T:
        cs = cs + jnp.concatenate(
            [jnp.zeros((k, _E), jnp.float32), cs[:_T - k, :]], axis=0)
        k *= 2
    rank = jnp.sum(cs * oh_f, axis=1, keepdims=True) - 1.0
    cnt = counts.astype(jnp.int32)
    aligned = ((cnt + (_TILE - 1)) // _TILE) * _TILE
    inc = aligned
    k = 1
    while k < _E:
        inc = inc + jnp.concatenate(
            [jnp.zeros((1, k), jnp.int32), inc[:, :_E - k]], axis=1)
        k *= 2
    start = inc - aligned
    dest = (jnp.sum(jnp.where(onehot, start, 0), axis=1, keepdims=True)
            + rank.astype(jnp.int32))
    dest_ref[...] = dest
    slots = lax.broadcasted_iota(jnp.int32, (_T, _NPAD), 1)
    oh = (slots == dest).astype(jnp.float32)
    tok = lax.broadcasted_iota(jnp.int32, (_T, 128), 0).astype(jnp.float32)
    lane = lax.broadcasted_iota(jnp.int32, (_T, 128), 1)
    cols = jnp.where(lane == 0, tok,
                     jnp.where(lane == 1, pmax,
                               jnp.where(lane == 2, 1.0, 0.0)))
    res = lax.dot_general(oh, cols, (((0,), (0,)), ((), ())),
                          preferred_element_type=jnp.float32)
    occ = res[:, 2:3]
    sidx = lax.broadcasted_iota(jnp.int32, (_NPAD, 128), 0)[:, :1]
    inv_ref[...] = jnp.where(occ > 0.5, res[:, :1].astype(jnp.int32),
                             jnp.bitwise_and(sidx, _T - 1))
    ps_ref[...] = res[:, 1:2]
    tvals = lax.broadcasted_iota(jnp.int32, (_NT, _E), 0) * _TILE
    eot = jnp.sum((tvals >= inc).astype(jnp.int32), axis=1, keepdims=True)
    eot_ref[...] = jnp.minimum(eot, _E - 1)
    act_ref[...] = (tvals[:, :1] < inc[:, _E - 1:_E]).astype(jnp.int32)


def _router_call(x2d, Wg, bg2d, interpret=False):
    return pl.pallas_call(
        _router_body,
        out_shape=(
            jax.ShapeDtypeStruct((_T, 1), jnp.int32),
            jax.ShapeDtypeStruct((_NPAD, 1), jnp.int32),
            jax.ShapeDtypeStruct((_NPAD, 1), jnp.float32),
            jax.ShapeDtypeStruct((_NT, 1), jnp.int32),
            jax.ShapeDtypeStruct((_NT, 1), jnp.int32),
            jax.ShapeDtypeStruct((1, 1), jnp.float32),
        ),
        interpret=interpret,
    )(x2d, Wg, bg2d)


_NHB = 4
_HB = _H // _NHB


def _moe_body(eot_ref, act_ref, x_ref, ps_ref, wu_ref, wv_ref, wdt_ref,
              out_ref):
    del eot_ref
    hb = pl.program_id(0)
    i = pl.program_id(1)

    @pl.when(act_ref[i] == 1)
    def _tile():
        x = x_ref[...].astype(jnp.bfloat16)
        u = lax.dot_general(x, wu_ref[0], (((1,), (1,)), ((), ())),
                            preferred_element_type=jnp.float32)
        v = lax.dot_general(x, wv_ref[0], (((1,), (1,)), ((), ())),
                            preferred_element_type=jnp.float32)
        s = (u * lax.logistic(u) * v).astype(jnp.bfloat16)
        y = lax.dot_general(s, wdt_ref[0], (((1,), (0,)), ((), ())),
                            preferred_element_type=jnp.float32)
        y = y * ps_ref[...]
        rows = pl.ds(i * _TILE, _TILE)

        @pl.when(hb == 0)
        def _init():
            out_ref[rows, :] = y

        @pl.when(hb != 0)
        def _acc():
            out_ref[rows, :] += y


def _grouped_call(eot_flat, act_flat, xs, ps_pad, Wu, Wv, WdT,
                  interpret=False):
    grid_spec = pltpu.PrefetchScalarGridSpec(
        num_scalar_prefetch=2,
        grid=(_NHB, _NT),
        in_specs=[
            pl.BlockSpec((_TILE, _D), lambda hb, i, eot, act: (i, 0)),
            pl.BlockSpec((_TILE, 1), lambda hb, i, eot, act: (i, 0)),
            pl.BlockSpec((1, _HB, _D), lambda hb, i, eot, act: (eot[i], hb, 0)),
            pl.BlockSpec((1, _HB, _D), lambda hb, i, eot, act: (eot[i], hb, 0)),
            pl.BlockSpec((1, _HB, _D), lambda hb, i, eot, act: (eot[i], hb, 0)),
        ],
        out_specs=pl.BlockSpec((_NPAD, _D), lambda hb, i, eot, act: (0, 0)),
    )
    return pl.pallas_call(
        _moe_body,
        grid_spec=grid_spec,
        out_shape=jax.ShapeDtypeStruct((_NPAD, _D), jnp.float32),
        compiler_params=pltpu.CompilerParams(
            dimension_semantics=("arbitrary", "arbitrary")),
        interpret=interpret,
    )(eot_flat, act_flat, xs, ps_pad, Wu, Wv, WdT)


def _sc_gather_rows(table, idx, nrows):
    rpw = nrows // _NW
    mesh = plsc.VectorSubcoreMesh(core_axis_name="c", subcore_axis_name="s")

    @functools.partial(
        pl.kernel,
        out_type=jax.ShapeDtypeStruct((nrows, _D), jnp.float32),
        mesh=mesh,
        scratch_types=[
            pltpu.VMEM((rpw,), jnp.int32),
            pltpu.VMEM((rpw, _D), jnp.float32),
            pltpu.SemaphoreType.DMA,
        ],
    )
    def gather_k(idx_hbm, table_hbm, out_hbm, idx_v, rows_v, sem):
        wid = lax.axis_index("s") * 2 + lax.axis_index("c")
        base = wid * rpw
        pltpu.sync_copy(idx_hbm.at[pl.ds(base, rpw)], idx_v)
        pltpu.async_copy(table_hbm.at[idx_v], rows_v, sem).wait()
        pltpu.sync_copy(rows_v, out_hbm.at[pl.ds(base, rpw)])

    return gather_k(idx, table)


def kernel(x, Wg, bg, Wu, Wv, Wd):
    x2d = x.reshape(_T, _D)
    dest, inv, ps_pad, eot, act, aux = _router_call(x2d, Wg, bg.reshape(1, _E))
    xs = _sc_gather_rows(x2d, inv.reshape(_NPAD), _NPAD)
    ys = _grouped_call(eot.reshape(_NT), act.reshape(_NT), xs, ps_pad,
                       Wu.astype(jnp.bfloat16), Wv.astype(jnp.bfloat16),
                       jnp.swapaxes(Wd, 1, 2).astype(jnp.bfloat16))
    y2d = _sc_gather_rows(ys, dest.reshape(_T), _T)
    return y2d.reshape(1, _T, _D), aux[0, 0]


# speedup vs baseline: 1.5965x; 1.5965x over previous
import functools

import jax
import jax.numpy as jnp
from jax import lax
from jax.experimental import pallas as pl
from jax.experimental.pallas import tpu as pltpu
from jax.experimental.pallas import tpu_sc as plsc

_T = 2048
_D = 1024
_E = 8
_H = 2752
_ALPHA = 0.05
_TILE = 256
_NPAD = _T + _E * _TILE
_NT = _NPAD // _TILE
_NW = 32


def _router_body(x_ref, wg_ref, bg_ref, dest_ref, inv_ref, ps_ref, eot_ref,
                 act_ref, aux_ref):
    x = x_ref[...]
    wg = wg_ref[...]
    logits = lax.dot_general(x, wg, (((1,), (1,)), ((), ())),
                             preferred_element_type=jnp.float32) + bg_ref[...]
    m = jnp.max(logits, axis=1, keepdims=True)
    ex = jnp.exp(logits - m)
    probs = ex / jnp.sum(ex, axis=1, keepdims=True)
    pmax = jnp.max(probs, axis=1, keepdims=True)
    lanes = lax.broadcasted_iota(jnp.int32, (_T, _E), 1)
    top1 = jnp.min(jnp.where(probs == pmax, lanes, _E), axis=1, keepdims=True)
    onehot = lanes == top1
    oh_f = onehot.astype(jnp.float32)
    counts = jnp.sum(oh_f, axis=0, keepdims=True)
    ce = jnp.sum(probs, axis=0, keepdims=True) * (1.0 / _T)
    me = counts * (1.0 / _T)
    aux_ref[...] = jnp.sum(me * ce, keepdims=True).reshape(1, 1) * (_ALPHA * _E)
    cs = oh_f
    k = 1
    while k < _T:
        cs = cs + jnp.concatenate(
            [jnp.zeros((k, _E), jnp.float32), cs[:_T - k, :]], axis=0)
        k *= 2
    rank = jnp.sum(cs * oh_f, axis=1, keepdims=True) - 1.0
    cnt = counts.astype(jnp.int32)
    aligned = ((cnt + (_TILE - 1)) // _TILE) * _TILE
    inc = aligned
    k = 1
    while k < _E:
        inc = inc + jnp.concatenate(
            [jnp.zeros((1, k), jnp.int32), inc[:, :_E - k]], axis=1)
        k *= 2
    start = inc - aligned
    dest = (jnp.sum(jnp.where(onehot, start, 0), axis=1, keepdims=True)
            + rank.astype(jnp.int32))
    dest_ref[...] = dest
    slots = lax.broadcasted_iota(jnp.int32, (_T, _NPAD), 1)
    oh = (slots == dest).astype(jnp.float32)
    tok = lax.broadcasted_iota(jnp.int32, (_T, 128), 0).astype(jnp.float32)
    lane = lax.broadcasted_iota(jnp.int32, (_T, 128), 1)
    cols = jnp.where(lane == 0, tok,
                     jnp.where(lane == 1, pmax,
                               jnp.where(lane == 2, 1.0, 0.0)))
    res = lax.dot_general(oh, cols, (((0,), (0,)), ((), ())),
                          preferred_element_type=jnp.float32)
    occ = res[:, 2:3]
    sidx = lax.broadcasted_iota(jnp.int32, (_NPAD, 128), 0)[:, :1]
    inv_ref[...] = jnp.where(occ > 0.5, res[:, :1].astype(jnp.int32),
                             jnp.bitwise_and(sidx, _T - 1))
    ps_ref[...] = res[:, 1:2]
    tvals = lax.broadcasted_iota(jnp.int32, (_NT, _E), 0) * _TILE
    eot = jnp.sum((tvals >= inc).astype(jnp.int32), axis=1, keepdims=True)
    eot_ref[...] = jnp.minimum(eot, _E - 1)
    act_ref[...] = (tvals[:, :1] < inc[:, _E - 1:_E]).astype(jnp.int32)


def _router_call(x2d, Wg, bg2d, interpret=False):
    return pl.pallas_call(
        _router_body,
        out_shape=(
            jax.ShapeDtypeStruct((_T, 1), jnp.int32),
            jax.ShapeDtypeStruct((_NPAD, 1), jnp.int32),
            jax.ShapeDtypeStruct((_NPAD, 1), jnp.float32),
            jax.ShapeDtypeStruct((_NT, 1), jnp.int32),
            jax.ShapeDtypeStruct((_NT, 1), jnp.int32),
            jax.ShapeDtypeStruct((1, 1), jnp.float32),
        ),
        interpret=interpret,
    )(x2d, Wg, bg2d)


_NHB = 4
_HB = _H // _NHB


def _moe_body(eot_ref, act_ref, x_ref, ps_ref, wu_ref, wv_ref, wdt_ref,
              out_ref):
    del eot_ref
    hb = pl.program_id(0)
    i = pl.program_id(1)

    @pl.when(act_ref[i] == 1)
    def _tile():
        x = x_ref[...]
        u = lax.dot_general(x, wu_ref[0], (((1,), (1,)), ((), ())),
                            preferred_element_type=jnp.float32)
        v = lax.dot_general(x, wv_ref[0], (((1,), (1,)), ((), ())),
                            preferred_element_type=jnp.float32)
        s = u * lax.logistic(u) * v
        y = lax.dot_general(s, wdt_ref[0], (((1,), (0,)), ((), ())),
                            preferred_element_type=jnp.float32)
        y = y * ps_ref[...]
        rows = pl.ds(i * _TILE, _TILE)

        @pl.when(hb == 0)
        def _init():
            out_ref[rows, :] = y

        @pl.when(hb != 0)
        def _acc():
            out_ref[rows, :] += y


def _grouped_call(eot_flat, act_flat, xs, ps_pad, Wu, Wv, WdT,
                  interpret=False):
    grid_spec = pltpu.PrefetchScalarGridSpec(
        num_scalar_prefetch=2,
        grid=(_NHB, _NT),
        in_specs=[
            pl.BlockSpec((_TILE, _D), lambda hb, i, eot, act: (i, 0)),
            pl.BlockSpec((_TILE, 1), lambda hb, i, eot, act: (i, 0)),
            pl.BlockSpec((1, _HB, _D), lambda hb, i, eot, act: (eot[i], hb, 0)),
            pl.BlockSpec((1, _HB, _D), lambda hb, i, eot, act: (eot[i], hb, 0)),
            pl.BlockSpec((1, _HB, _D), lambda hb, i, eot, act: (eot[i], hb, 0)),
        ],
        out_specs=pl.BlockSpec((_NPAD, _D), lambda hb, i, eot, act: (0, 0)),
    )
    return pl.pallas_call(
        _moe_body,
        grid_spec=grid_spec,
        out_shape=jax.ShapeDtypeStruct((_NPAD, _D), jnp.float32),
        compiler_params=pltpu.CompilerParams(
            dimension_semantics=("arbitrary", "arbitrary")),
        interpret=interpret,
    )(eot_flat, act_flat, xs, ps_pad, Wu, Wv, WdT)


def _sc_gather_rows(table, idx, nrows):
    rpw = nrows // _NW
    chunk = rpw
    while chunk > 96:
        chunk //= 2
    nck = rpw // chunk
    mesh = plsc.VectorSubcoreMesh(core_axis_name="c", subcore_axis_name="s")

    @functools.partial(
        pl.kernel,
        out_type=jax.ShapeDtypeStruct((nrows, _D), jnp.float32),
        mesh=mesh,
        scratch_types=[
            pltpu.VMEM((chunk,), jnp.int32),
            pltpu.VMEM((chunk, _D), jnp.float32),
            pltpu.SemaphoreType.DMA,
        ],
    )
    def gather_k(idx_hbm, table_hbm, out_hbm, idx_v, rows_v, sem):
        wid = lax.axis_index("s") * 2 + lax.axis_index("c")
        for c in range(nck):
            base = wid * rpw + c * chunk
            pltpu.sync_copy(idx_hbm.at[pl.ds(base, chunk)], idx_v)
            pltpu.async_copy(table_hbm.at[idx_v], rows_v, sem).wait()
            pltpu.sync_copy(rows_v, out_hbm.at[pl.ds(base, chunk)])

    return gather_k(idx, table)


def kernel(x, Wg, bg, Wu, Wv, Wd):
    x2d = x.reshape(_T, _D)
    dest, inv, ps_pad, eot, act, aux = _router_call(x2d, Wg, bg.reshape(1, _E))
    xs = _sc_gather_rows(x2d, inv.reshape(_NPAD), _NPAD)
    ys = _grouped_call(eot.reshape(_NT), act.reshape(_NT), xs, ps_pad,
                       Wu, Wv, jnp.swapaxes(Wd, 1, 2))
    y2d = _sc_gather_rows(ys, dest.reshape(_T), _T)
    return y2d.reshape(1, _T, _D), aux[0, 0]


# speedup vs baseline: 1.7528x; 1.0979x over previous
import functools

import jax
import jax.numpy as jnp
from jax import lax
from jax.experimental import pallas as pl
from jax.experimental.pallas import tpu as pltpu
from jax.experimental.pallas import tpu_sc as plsc

_T = 2048
_D = 1024
_E = 8
_H = 2752
_ALPHA = 0.05
_TILE = 256
_NPAD = _T + _E * _TILE
_NT = _NPAD // _TILE
_NW = 32


def _router_body(x_ref, wg_ref, bg_ref, dest_ref, inv_ref, ps_ref, eot_ref,
                 act_ref, aux_ref):
    x = x_ref[...]
    wg = wg_ref[...]
    logits = lax.dot_general(x, wg, (((1,), (1,)), ((), ())),
                             preferred_element_type=jnp.float32) + bg_ref[...]
    m = jnp.max(logits, axis=1, keepdims=True)
    ex = jnp.exp(logits - m)
    probs = ex / jnp.sum(ex, axis=1, keepdims=True)
    pmax = jnp.max(probs, axis=1, keepdims=True)
    lanes = lax.broadcasted_iota(jnp.int32, (_T, _E), 1)
    top1 = jnp.min(jnp.where(probs == pmax, lanes, _E), axis=1, keepdims=True)
    onehot = lanes == top1
    oh_f = onehot.astype(jnp.float32)
    counts = jnp.sum(oh_f, axis=0, keepdims=True)
    ce = jnp.sum(probs, axis=0, keepdims=True) * (1.0 / _T)
    me = counts * (1.0 / _T)
    aux_ref[...] = jnp.sum(me * ce, keepdims=True).reshape(1, 1) * (_ALPHA * _E)
    cs = oh_f
    k = 1
    while k < _T:
        cs = cs + jnp.concatenate(
            [jnp.zeros((k, _E), jnp.float32), cs[:_T - k, :]], axis=0)
        k *= 2
    rank = jnp.sum(cs * oh_f, axis=1, keepdims=True) - 1.0
    cnt = counts.astype(jnp.int32)
    aligned = ((cnt + (_TILE - 1)) // _TILE) * _TILE
    inc = aligned
    k = 1
    while k < _E:
        inc = inc + jnp.concatenate(
            [jnp.zeros((1, k), jnp.int32), inc[:, :_E - k]], axis=1)
        k *= 2
    start = inc - aligned
    dest = (jnp.sum(jnp.where(onehot, start, 0), axis=1, keepdims=True)
            + rank.astype(jnp.int32))
    dest_ref[...] = dest
    slots = lax.broadcasted_iota(jnp.int32, (_T, _NPAD), 1)
    oh = (slots == dest).astype(jnp.float32)
    tok = lax.broadcasted_iota(jnp.int32, (_T, 128), 0).astype(jnp.float32)
    lane = lax.broadcasted_iota(jnp.int32, (_T, 128), 1)
    cols = jnp.where(lane == 0, tok,
                     jnp.where(lane == 1, pmax,
                               jnp.where(lane == 2, 1.0, 0.0)))
    res = lax.dot_general(oh, cols, (((0,), (0,)), ((), ())),
                          preferred_element_type=jnp.float32)
    occ = res[:, 2:3]
    sidx = lax.broadcasted_iota(jnp.int32, (_NPAD, 128), 0)[:, :1]
    inv_ref[...] = jnp.where(occ > 0.5, res[:, :1].astype(jnp.int32),
                             jnp.bitwise_and(sidx, _T - 1))
    ps_ref[...] = res[:, 1:2]
    tvals = lax.broadcasted_iota(jnp.int32, (_NT, _E), 0) * _TILE
    eot = jnp.sum((tvals >= inc).astype(jnp.int32), axis=1, keepdims=True)
    eot_ref[...] = jnp.minimum(eot, _E - 1)
    act_ref[...] = (tvals[:, :1] < inc[:, _E - 1:_E]).astype(jnp.int32)


def _router_call(x2d, Wg, bg2d, interpret=False):
    return pl.pallas_call(
        _router_body,
        out_shape=(
            jax.ShapeDtypeStruct((_T, 1), jnp.int32),
            jax.ShapeDtypeStruct((_NPAD, 1), jnp.int32),
            jax.ShapeDtypeStruct((_NPAD, 1), jnp.float32),
            jax.ShapeDtypeStruct((_NT, 1), jnp.int32),
            jax.ShapeDtypeStruct((_NT, 1), jnp.int32),
            jax.ShapeDtypeStruct((1, 1), jnp.float32),
        ),
        interpret=interpret,
    )(x2d, Wg, bg2d)


_NHB = 2
_HB = _H // _NHB


def _moe_body(eot_ref, act_ref, x_ref, ps_ref, wu_ref, wv_ref, wdt_ref,
              out_ref):
    del eot_ref
    hb = pl.program_id(0)
    i = pl.program_id(1)

    @pl.when(act_ref[i] == 1)
    def _tile():
        x = x_ref[...]
        u = lax.dot_general(x, wu_ref[0], (((1,), (1,)), ((), ())),
                            preferred_element_type=jnp.float32)
        v = lax.dot_general(x, wv_ref[0], (((1,), (1,)), ((), ())),
                            preferred_element_type=jnp.float32)
        s = u * lax.logistic(u) * v
        y = lax.dot_general(s, wdt_ref[0], (((1,), (0,)), ((), ())),
                            preferred_element_type=jnp.float32)
        y = y * ps_ref[...]
        rows = pl.ds(i * _TILE, _TILE)

        @pl.when(hb == 0)
        def _init():
            out_ref[rows, :] = y

        @pl.when(hb != 0)
        def _acc():
            out_ref[rows, :] += y


def _grouped_call(eot_flat, act_flat, xs, ps_pad, Wu, Wv, WdT,
                  interpret=False):
    grid_spec = pltpu.PrefetchScalarGridSpec(
        num_scalar_prefetch=2,
        grid=(_NHB, _NT),
        in_specs=[
            pl.BlockSpec((_TILE, _D), lambda hb, i, eot, act: (i, 0)),
            pl.BlockSpec((_TILE, 1), lambda hb, i, eot, act: (i, 0)),
            pl.BlockSpec((1, _HB, _D), lambda hb, i, eot, act: (eot[i], hb, 0)),
            pl.BlockSpec((1, _HB, _D), lambda hb, i, eot, act: (eot[i], hb, 0)),
            pl.BlockSpec((1, _HB, _D), lambda hb, i, eot, act: (eot[i], hb, 0)),
        ],
        out_specs=pl.BlockSpec((_NPAD, _D), lambda hb, i, eot, act: (0, 0)),
    )
    return pl.pallas_call(
        _moe_body,
        grid_spec=grid_spec,
        out_shape=jax.ShapeDtypeStruct((_NPAD, _D), jnp.float32),
        compiler_params=pltpu.CompilerParams(
            dimension_semantics=("arbitrary", "arbitrary")),
        interpret=interpret,
    )(eot_flat, act_flat, xs, ps_pad, Wu, Wv, WdT)


def _sc_gather_rows(table, idx, nrows):
    rpw = nrows // _NW
    chunk = rpw
    while chunk > 96:
        chunk //= 2
    nck = rpw // chunk
    mesh = plsc.VectorSubcoreMesh(core_axis_name="c", subcore_axis_name="s")

    @functools.partial(
        pl.kernel,
        out_type=jax.ShapeDtypeStruct((nrows, _D), jnp.float32),
        mesh=mesh,
        scratch_types=[
            pltpu.VMEM((chunk,), jnp.int32),
            pltpu.VMEM((chunk, _D), jnp.float32),
            pltpu.SemaphoreType.DMA,
        ],
    )
    def gather_k(idx_hbm, table_hbm, out_hbm, idx_v, rows_v, sem):
        wid = lax.axis_index("s") * 2 + lax.axis_index("c")
        for c in range(nck):
            base = wid * rpw + c * chunk
            pltpu.sync_copy(idx_hbm.at[pl.ds(base, chunk)], idx_v)
            pltpu.async_copy(table_hbm.at[idx_v], rows_v, sem).wait()
            pltpu.sync_copy(rows_v, out_hbm.at[pl.ds(base, chunk)])

    return gather_k(idx, table)


def kernel(x, Wg, bg, Wu, Wv, Wd):
    x2d = x.reshape(_T, _D)
    dest, inv, ps_pad, eot, act, aux = _router_call(x2d, Wg, bg.reshape(1, _E))
    xs = _sc_gather_rows(x2d, inv.reshape(_NPAD), _NPAD)
    ys = _grouped_call(eot.reshape(_NT), act.reshape(_NT), xs, ps_pad,
                       Wu, Wv, jnp.swapaxes(Wd, 1, 2))
    y2d = _sc_gather_rows(ys, dest.reshape(_T), _T)
    return y2d.reshape(1, _T, _D), aux[0, 0]
